# Initial kernel scaffold; baseline (speedup 1.0000x reference)
import functools

import jax
import jax.numpy as jnp
from jax import lax
from jax.experimental import pallas as pl
from jax.experimental.pallas import tpu as pltpu
from jax.experimental.pallas import tpu_sc as plsc

ALPHA = 0.2
LANES = 16
NC = 2
NS = 16
NW = NC * NS
CHUNK = 128
DW = 144


def _proj_body(x_ref, w_ref, a2_ref, h_ref, st_ref):
    h = jnp.dot(x_ref[...], w_ref[...], preferred_element_type=jnp.float32)
    h_ref[...] = h
    st_ref[...] = lax.dot_general(
        a2_ref[...], h, (((0,), (1,)), ((), ())),
        preferred_element_type=jnp.float32)


def _project(x_p, W, a2, NP, D):
    BLK = 512
    grid = NP // BLK
    return pl.pallas_call(
        _proj_body,
        grid=(grid,),
        in_specs=[
            pl.BlockSpec((BLK, x_p.shape[1]), lambda i: (i, 0)),
            pl.BlockSpec((x_p.shape[1], D), lambda i: (0, 0)),
            pl.BlockSpec((D, 2), lambda i: (0, 0)),
        ],
        out_specs=[
            pl.BlockSpec((BLK, D), lambda i: (i, 0)),
            pl.BlockSpec((2, BLK), lambda i: (0, i)),
        ],
        out_shape=[
            jax.ShapeDtypeStruct((NP, D), jnp.float32),
            jax.ShapeDtypeStruct((2, NP), jnp.float32),
        ],
    )(x_p, W, a2)


def _sc_body(E, EPT, NP, D, NCH,
             h_hbm, st_hbm, src_hbm, dst_hbm, out_hbm,
             s_v, t_v, srcw_v, dst_v, ee_v, hbuf_v, sbuf_v, zrow_v,
             acc_sh, gsem):
    cid = lax.axis_index("c")
    sid = lax.axis_index("s")
    wid = sid * NC + cid

    rows_per_tile = NP // NS

    @pl.loop(0, CHUNK)
    def _(r):
        for g in range(DW // LANES):
            zrow_v[r, pl.ds(g * LANES, LANES)] = jnp.zeros(
                (LANES,), jnp.float32)

    @pl.loop(0, rows_per_tile // CHUNK)
    def _(b):
        pltpu.sync_copy(
            zrow_v, acc_sh.at[pl.ds(sid * rows_per_tile + b * CHUNK, CHUNK)])

    pltpu.sync_copy(st_hbm.at[0], s_v)
    pltpu.sync_copy(st_hbm.at[1], t_v)
    pltpu.sync_copy(src_hbm.at[wid], srcw_v)
    pltpu.sync_copy(dst_hbm.at[wid], dst_v)

    base = wid * EPT

    @pl.loop(0, NCH)
    def _(j):
        for g in range(CHUNK // LANES):
            off = j * CHUNK + g * LANES
            srcv = srcw_v[j, 0, pl.ds(g * LANES, LANES)]
            dstv = dst_v[pl.ds(off, LANES)]
            sv = plsc.load_gather(s_v, [srcv])
            tv = plsc.load_gather(t_v, [dstv])
            logit = sv + tv
            lrelu = jnp.where(logit >= 0.0, logit, ALPHA * logit)
            ee = jnp.exp(-lrelu)
            pos = base + off + lax.iota(jnp.int32, LANES)
            ee = jnp.where(pos < E, ee, 0.0)
            ee_v[pl.ds(off, LANES)] = ee

    plsc.subcore_barrier()

    @pl.loop(0, NCH)
    def _(j):
        pltpu.async_copy(
            h_hbm.at[dst_v.at[pl.ds(j * CHUNK, CHUNK)]], hbuf_v, gsem,
        ).wait()

        @pl.loop(0, CHUNK)
        def _(k):
            idx = jnp.full((LANES,), j * CHUNK + k, jnp.int32)
            eeb = plsc.load_gather(ee_v, [idx])
            for g in range(D // LANES):
                sbuf_v[0, k, pl.ds(g * LANES, LANES)] = (
                    hbuf_v[k, pl.ds(g * LANES, LANES)] * eeb)
            lane = lax.iota(jnp.int32, LANES)
            sbuf_v[0, k, pl.ds(D, LANES)] = jnp.where(lane == 0, eeb, 0.0)

        pltpu.sync_copy(sbuf_v, acc_sh.at[srcw_v.at[j]], add=True)

    plsc.subcore_barrier()

    pltpu.sync_copy(
        acc_sh.at[pl.ds(sid * rows_per_tile, rows_per_tile)],
        out_hbm.at[cid, pl.ds(sid * rows_per_tile, rows_per_tile)])


def _sc_aggregate(h, st, src_w, dst_w, E, EPT, NP, D, NCH):
    mesh = plsc.VectorSubcoreMesh(
        core_axis_name="c", subcore_axis_name="s",
        num_cores=NC, num_subcores=NS)
    kern = pl.kernel(
        functools.partial(_sc_body, E, EPT, NP, D, NCH),
        out_type=jax.ShapeDtypeStruct((NC, NP, DW), jnp.float32),
        mesh=mesh,
        scratch_types=[
            pltpu.VMEM((NP,), jnp.float32),
            pltpu.VMEM((NP,), jnp.float32),
            pltpu.VMEM((NCH, 1, CHUNK), jnp.int32),
            pltpu.VMEM((EPT,), jnp.int32),
            pltpu.VMEM((EPT,), jnp.float32),
            pltpu.VMEM((CHUNK, D), jnp.float32),
            pltpu.VMEM((1, CHUNK, DW), jnp.float32),
            pltpu.VMEM((CHUNK, DW), jnp.float32),
            pltpu.VMEM_SHARED((NP, DW), jnp.float32),
            pltpu.SemaphoreType.DMA,
        ],
    )
    return kern(h, st, src_w, dst_w)


def _combine_body(p_ref, o_ref):
    p = p_ref[...]
    tot = p[0] + p[1]
    hh = tot[:, :128]
    r = tot[:, 128:129]
    v = hh / r
    o_ref[...] = jnp.where(v > 0.0, v, jnp.expm1(v))


def _combine(parts, NP, D):
    BLK = 512
    return pl.pallas_call(
        _combine_body,
        grid=(NP // BLK,),
        in_specs=[pl.BlockSpec((NC, BLK, DW), lambda i: (0, i, 0))],
        out_specs=pl.BlockSpec((BLK, D), lambda i: (i, 0)),
        out_shape=jax.ShapeDtypeStruct((NP, D), jnp.float32),
    )(parts)


def kernel(input, edge, W, a):
    N, DIN = input.shape
    D = W.shape[1]
    E = edge.shape[1]

    NP = ((N + 511) // 512) * 512
    EPT = ((E + NW * CHUNK - 1) // (NW * CHUNK)) * CHUNK
    NCH = EPT // CHUNK

    x_p = jnp.pad(input, ((0, NP - N), (0, 0)))
    a2 = jnp.stack([a[0, :D], a[0, D:]], axis=1)

    pad = NW * EPT - E
    src_w = jnp.pad(edge[0], (0, pad)).reshape(NW, NCH, 1, CHUNK)
    dst_w = jnp.pad(edge[1], (0, pad)).reshape(NW, EPT)

    h, st = _project(x_p, W, a2, NP, D)
    parts = _sc_aggregate(h, st, src_w, dst_w, E, EPT, NP, D, NCH)
    out = _combine(parts, NP, D)
    return out[:N]


# speedup vs baseline: 6.0711x; 6.0711x over previous
import functools

import jax
import jax.numpy as jnp
from jax import lax
from jax.experimental import pallas as pl
from jax.experimental.pallas import tpu as pltpu
from jax.experimental.pallas import tpu_sc as plsc

ALPHA = 0.2
LANES = 16
NC = 2
NS = 16
NW = NC * NS
CHUNK = 128


def _proj_body(x_ref, w_ref, a2_ref, h_ref, st_ref):
    h = jnp.dot(x_ref[...], w_ref[...], preferred_element_type=jnp.float32)
    h_ref[...] = h
    st_ref[...] = lax.dot_general(
        a2_ref[...], h, (((0,), (1,)), ((), ())),
        preferred_element_type=jnp.float32)


def _project(x_p, W, a2, NP, D):
    BLK = 512
    grid = NP // BLK
    return pl.pallas_call(
        _proj_body,
        grid=(grid,),
        in_specs=[
            pl.BlockSpec((BLK, x_p.shape[1]), lambda i: (i, 0)),
            pl.BlockSpec((x_p.shape[1], D), lambda i: (0, 0)),
            pl.BlockSpec((D, 2), lambda i: (0, 0)),
        ],
        out_specs=[
            pl.BlockSpec((BLK, D), lambda i: (i, 0)),
            pl.BlockSpec((2, BLK), lambda i: (0, i)),
        ],
        out_shape=[
            jax.ShapeDtypeStruct((NP, D), jnp.float32),
            jax.ShapeDtypeStruct((2, NP), jnp.float32),
        ],
    )(x_p, W, a2)


def _ee_kernel(st, src_w, dst_w, E, EPT, NP, NR):
    mesh = plsc.VectorSubcoreMesh(
        core_axis_name="c", subcore_axis_name="s",
        num_cores=NC, num_subcores=NS)
    kern = pl.kernel(
        functools.partial(_ee_body2, E, EPT, NP, NR),
        out_type=(
            jax.ShapeDtypeStruct((NW, EPT), jnp.float32),
            jax.ShapeDtypeStruct((NC, NR, 128), jnp.float32),
        ),
        mesh=mesh,
        compiler_params=pltpu.CompilerParams(needs_layout_passes=False),
        scratch_types=[
            pltpu.VMEM((NP,), jnp.float32),
            pltpu.VMEM((NP,), jnp.float32),
            pltpu.VMEM((EPT,), jnp.int32),
            pltpu.VMEM((EPT,), jnp.int32),
            pltpu.VMEM((EPT,), jnp.float32),
            pltpu.VMEM((NR, 128), jnp.float32),
            pltpu.VMEM((NR,), jnp.int32),
            pltpu.VMEM_SHARED((NR, 128), jnp.float32),
        ],
    )
    return kern(st, src_w, dst_w)


def _ee_body2(E, EPT, NP, NR,
              st_hbm, src_hbm, dst_hbm, ee_hbm, rs_hbm,
              s_v, t_v, src_v, dst_v, ee_v, rsum_v, iota_v, rs_sh):
    cid = lax.axis_index("c")
    sid = lax.axis_index("s")
    wid = sid * NC + cid

    zero = jnp.zeros((LANES,), jnp.float32)

    @pl.loop(0, NR)
    def _(r):
        for g in range(128 // LANES):
            rsum_v[r, pl.ds(g * LANES, LANES)] = zero

    @pl.loop(0, NR // LANES)
    def _(g):
        iota_v[pl.ds(g * LANES, LANES)] = (
            g * LANES + lax.iota(jnp.int32, LANES))

    @pl.when(sid == 0)
    def _():
        pltpu.sync_copy(rsum_v, rs_sh)

    pltpu.sync_copy(st_hbm.at[0], s_v)
    pltpu.sync_copy(st_hbm.at[1], t_v)
    pltpu.sync_copy(src_hbm.at[wid], src_v)
    pltpu.sync_copy(dst_hbm.at[wid], dst_v)

    plsc.subcore_barrier()

    base = wid * EPT

    @pl.loop(0, EPT // LANES)
    def _(i):
        off = i * LANES
        srcv = src_v[pl.ds(off, LANES)]
        dstv = dst_v[pl.ds(off, LANES)]
        sv = plsc.load_gather(s_v, [srcv])
        tv = plsc.load_gather(t_v, [dstv])
        logit = sv + tv
        lrelu = jnp.where(logit >= 0.0, logit, ALPHA * logit)
        ee = jnp.exp(-lrelu)
        pos = base + off + lax.iota(jnp.int32, LANES)
        ee = jnp.where(pos < E, ee, 0.0)
        ee_v[pl.ds(off, LANES)] = ee
        rhi = lax.shift_right_logical(srcv, 7)
        rlo = jnp.bitwise_and(srcv, 127)
        plsc.addupdate_scatter(rsum_v, [rhi, rlo], ee)

    pltpu.sync_copy(ee_v, ee_hbm.at[wid])

    pltpu.sync_copy(rsum_v, rs_sh.at[iota_v], add=True)

    plsc.subcore_barrier()

    @pl.when(sid == 0)
    def _():
        pltpu.sync_copy(rs_sh, rs_hbm.at[cid])


def _agg_body(EPT, NP, D, NCH,
              h_hbm, srci_hbm, dsti_hbm, ee_hbm, out_hbm,
              src_v, dst_v, ee_v, hbuf, sem, acc_sh):
    cid = lax.axis_index("c")
    sid = lax.axis_index("s")
    wid = sid * NC + cid

    rows_per_tile = NP // NS

    zero = jnp.zeros((LANES,), jnp.float32)

    @pl.loop(0, CHUNK)
    def _(r):
        for g in range(D // LANES):
            hbuf[r, pl.ds(g * LANES, LANES)] = zero

    @pl.loop(0, rows_per_tile // CHUNK)
    def _(b):
        pltpu.sync_copy(
            hbuf, acc_sh.at[pl.ds(sid * rows_per_tile + b * CHUNK, CHUNK)])

    pltpu.sync_copy(srci_hbm.at[wid], src_v)
    pltpu.sync_copy(dsti_hbm.at[wid], dst_v)
    pltpu.sync_copy(ee_hbm.at[wid], ee_v)

    plsc.subcore_barrier()

    @pl.loop(0, NCH)
    def _(j):
        pltpu.async_copy(
            h_hbm.at[dst_v.at[pl.ds(j * CHUNK, CHUNK)]], hbuf, sem,
        ).wait()

        @pl.loop(0, CHUNK)
        def _(k):
            idx = j * CHUNK + k + jnp.zeros((LANES,), jnp.int32)
            eeb = plsc.load_gather(ee_v, [idx])
            for g in range(D // LANES):
                hbuf[k, pl.ds(g * LANES, LANES)] = (
                    hbuf[k, pl.ds(g * LANES, LANES)] * eeb)

        pltpu.sync_copy(hbuf, acc_sh.at[src_v.at[j]], add=True)

    plsc.subcore_barrier()

    pltpu.sync_copy(
        acc_sh.at[pl.ds(sid * rows_per_tile, rows_per_tile)],
        out_hbm.at[cid, pl.ds(sid * rows_per_tile, rows_per_tile)])


def _agg_kernel(h, src_i, dst_i, ee, EPT, NP, D, NCH):
    mesh = plsc.VectorSubcoreMesh(
        core_axis_name="c", subcore_axis_name="s",
        num_cores=NC, num_subcores=NS)
    kern = pl.kernel(
        functools.partial(_agg_body, EPT, NP, D, NCH),
        out_type=jax.ShapeDtypeStruct((NC, NP, D), jnp.float32),
        mesh=mesh,
        compiler_params=pltpu.CompilerParams(needs_layout_passes=False),
        scratch_types=[
            pltpu.VMEM((NCH, CHUNK), jnp.int32),
            pltpu.VMEM((EPT,), jnp.int32),
            pltpu.VMEM((EPT,), jnp.float32),
            pltpu.VMEM((CHUNK, D), jnp.float32),
            pltpu.SemaphoreType.DMA,
            pltpu.VMEM_SHARED((NP, D), jnp.float32),
        ],
    )
    return kern(h, src_i, dst_i, ee)


def _combine_body(p_ref, r_ref, o_ref):
    p = p_ref[...]
    r = r_ref[...]
    tot = p[0] + p[1]
    rr = r[0] + r[1]
    v = tot / rr
    o_ref[...] = jnp.where(v > 0.0, v, jnp.exp(jnp.minimum(v, 0.0)) - 1.0)


def _combine(parts, rs3, NP, D):
    BLK = 512
    return pl.pallas_call(
        _combine_body,
        grid=(NP // BLK,),
        in_specs=[
            pl.BlockSpec((NC, BLK, D), lambda i: (0, i, 0)),
            pl.BlockSpec((NC, BLK, 1), lambda i: (0, i, 0)),
        ],
        out_specs=pl.BlockSpec((BLK, D), lambda i: (i, 0)),
        out_shape=jax.ShapeDtypeStruct((NP, D), jnp.float32),
    )(parts, rs3)


def kernel(input, edge, W, a):
    N, DIN = input.shape
    D = W.shape[1]
    E = edge.shape[1]

    NP = ((N + 511) // 512) * 512
    NR = NP // 128
    EPT = ((E + NW * CHUNK - 1) // (NW * CHUNK)) * CHUNK
    NCH = EPT // CHUNK

    x_p = jnp.pad(input, ((0, NP - N), (0, 0)))
    a2 = jnp.stack([a[0, :D], a[0, D:]], axis=1)

    pad = NW * EPT - E
    src_f = jnp.pad(edge[0], (0, pad)).reshape(NW, EPT)
    dst_f = jnp.pad(edge[1], (0, pad)).reshape(NW, EPT)
    src_c = src_f.reshape(NW, NCH, CHUNK)

    h, st = _project(x_p, W, a2, NP, D)
    ee, rsp = _ee_kernel(st, src_f, dst_f, E, EPT, NP, NR)
    parts = _agg_kernel(h, src_c, dst_f, ee, EPT, NP, D, NCH)
    rs3 = rsp.reshape(NC, NP, 1)
    out = _combine(parts, rs3, NP, D)
    return out[:N]


# speedup vs baseline: 8.7470x; 1.4408x over previous
import functools

import jax
import jax.numpy as jnp
from jax import lax
from jax.experimental import pallas as pl
from jax.experimental.pallas import tpu as pltpu
from jax.experimental.pallas import tpu_sc as plsc

ALPHA = 0.2
LANES = 16
NC = 2
NS = 16
NW = NC * NS
CHUNK = 128


def _proj_body(x_ref, w_ref, a2_ref, h_ref, st_ref):
    h = jnp.dot(x_ref[...], w_ref[...], preferred_element_type=jnp.float32)
    h_ref[...] = h
    st_ref[...] = lax.dot_general(
        a2_ref[...], h, (((0,), (1,)), ((), ())),
        preferred_element_type=jnp.float32)


def _project(x_p, W, a2, NP, D):
    BLK = 512
    grid = NP // BLK
    return pl.pallas_call(
        _proj_body,
        grid=(grid,),
        in_specs=[
            pl.BlockSpec((BLK, x_p.shape[1]), lambda i: (i, 0)),
            pl.BlockSpec((x_p.shape[1], D), lambda i: (0, 0)),
            pl.BlockSpec((D, 2), lambda i: (0, 0)),
        ],
        out_specs=[
            pl.BlockSpec((BLK, D), lambda i: (i, 0)),
            pl.BlockSpec((2, BLK), lambda i: (0, i)),
        ],
        out_shape=[
            jax.ShapeDtypeStruct((NP, D), jnp.float32),
            jax.ShapeDtypeStruct((2, NP), jnp.float32),
        ],
    )(x_p, W, a2)


def _ee_kernel(st, src_w, dst_w, E, EPT, NP, NR):
    mesh = plsc.VectorSubcoreMesh(
        core_axis_name="c", subcore_axis_name="s",
        num_cores=NC, num_subcores=NS)
    kern = pl.kernel(
        functools.partial(_ee_body2, E, EPT, NP, NR),
        out_type=(
            jax.ShapeDtypeStruct((NW, EPT), jnp.float32),
            jax.ShapeDtypeStruct((NC, NR, 128), jnp.float32),
        ),
        mesh=mesh,
        compiler_params=pltpu.CompilerParams(needs_layout_passes=False),
        scratch_types=[
            pltpu.VMEM((NP,), jnp.float32),
            pltpu.VMEM((NP,), jnp.float32),
            pltpu.VMEM((EPT,), jnp.int32),
            pltpu.VMEM((EPT,), jnp.int32),
            pltpu.VMEM((EPT,), jnp.float32),
            pltpu.VMEM((NR, 128), jnp.float32),
            pltpu.VMEM((NR,), jnp.int32),
            pltpu.VMEM_SHARED((NR, 128), jnp.float32),
        ],
    )
    return kern(st, src_w, dst_w)


def _ee_body2(E, EPT, NP, NR,
              st_hbm, src_hbm, dst_hbm, ee_hbm, rs_hbm,
              s_v, t_v, src_v, dst_v, ee_v, rsum_v, iota_v, rs_sh):
    cid = lax.axis_index("c")
    sid = lax.axis_index("s")
    wid = sid * NC + cid

    zero = jnp.zeros((LANES,), jnp.float32)

    @pl.loop(0, NR)
    def _(r):
        for g in range(128 // LANES):
            rsum_v[r, pl.ds(g * LANES, LANES)] = zero

    @pl.loop(0, NR // LANES)
    def _(g):
        iota_v[pl.ds(g * LANES, LANES)] = (
            g * LANES + lax.iota(jnp.int32, LANES))

    @pl.when(sid == 0)
    def _():
        pltpu.sync_copy(rsum_v, rs_sh)

    pltpu.sync_copy(st_hbm.at[0], s_v)
    pltpu.sync_copy(st_hbm.at[1], t_v)
    pltpu.sync_copy(src_hbm.at[wid], src_v)
    pltpu.sync_copy(dst_hbm.at[wid], dst_v)

    plsc.subcore_barrier()

    base = wid * EPT

    @pl.loop(0, EPT // LANES)
    def _(i):
        off = i * LANES
        srcv = src_v[pl.ds(off, LANES)]
        dstv = dst_v[pl.ds(off, LANES)]
        sv = plsc.load_gather(s_v, [srcv])
        tv = plsc.load_gather(t_v, [dstv])
        logit = sv + tv
        lrelu = jnp.where(logit >= 0.0, logit, ALPHA * logit)
        ee = jnp.exp(-lrelu)
        pos = base + off + lax.iota(jnp.int32, LANES)
        ee = jnp.where(pos < E, ee, 0.0)
        ee_v[pl.ds(off, LANES)] = ee
        rhi = lax.shift_right_logical(srcv, 7)
        rlo = jnp.bitwise_and(srcv, 127)
        plsc.addupdate_scatter(rsum_v, [rhi, rlo], ee)

    pltpu.sync_copy(ee_v, ee_hbm.at[wid])

    pltpu.sync_copy(rsum_v, rs_sh.at[iota_v], add=True)

    plsc.subcore_barrier()

    @pl.when(sid == 0)
    def _():
        pltpu.sync_copy(rs_sh, rs_hbm.at[cid])


NBUF = 2


def _agg_body(EPT, NP, D, NCH,
              h_hbm, srci_hbm, dsti_hbm, ee_hbm, out_hbm,
              src_v, db0, db1, eb0, eb1,
              hb0, hb1,
              m0, m1, e0, e1, g0, g1, s0, s1,
              acc_sh):
    cid = lax.axis_index("c")
    sid = lax.axis_index("s")
    wid = sid * NC + cid
    dbs = [db0, db1]
    ebs = [eb0, eb1]
    hbs = [hb0, hb1]
    msems = [m0, m1]
    esems = [e0, e1]
    gsems = [g0, g1]
    ssems = [s0, s1]

    rows_per_tile = NP // NS

    zero = jnp.zeros((LANES,), jnp.float32)

    @pl.loop(0, CHUNK)
    def _(r):
        for g in range(D // LANES):
            hb0[r, pl.ds(g * LANES, LANES)] = zero

    @pl.loop(0, rows_per_tile // CHUNK)
    def _(b):
        pltpu.sync_copy(
            hb0, acc_sh.at[pl.ds(sid * rows_per_tile + b * CHUNK, CHUNK)])

    pltpu.sync_copy(srci_hbm.at[wid], src_v)

    plsc.subcore_barrier()

    def issue_meta(j, b):
        pltpu.async_copy(dsti_hbm.at[wid, j], dbs[b], msems[b])
        pltpu.async_copy(ee_hbm.at[wid, j], ebs[b], esems[b])

    def wait_meta(b):
        pltpu.make_async_copy(dsti_hbm.at[0, 0], dbs[b], msems[b]).wait()
        pltpu.make_async_copy(ee_hbm.at[0, 0], ebs[b], esems[b]).wait()

    def issue_gather(j, b):
        pltpu.async_copy(h_hbm.at[dbs[b].at[0]], hbs[b], gsems[b])

    def drain(sem, b):
        pltpu.make_async_copy(h_hbm.at[pl.ds(0, CHUNK)], hbs[b], sem).wait()

    issue_meta(0, 0)
    issue_meta(1, 1)
    wait_meta(0)
    issue_gather(0, 0)

    @pl.loop(0, NCH)
    def _(j):
        for q in range(NBUF):
            @pl.when(lax.rem(j, 2) == q)
            def _():
                nq = 1 - q
                hb = hbs[q]
                eb = ebs[q]

                @pl.when(j + 1 < NCH)
                def _():
                    @pl.when(j >= 1)
                    def _():
                        drain(ssems[nq], nq)
                    wait_meta(nq)
                    issue_gather(j + 1, nq)

                drain(gsems[q], q)

                z16 = jnp.zeros((LANES,), jnp.int32)

                @pl.loop(0, CHUNK)
                def _(k):
                    eeb = plsc.load_gather(eb, [z16, k + z16])
                    for g in range(D // LANES):
                        hb[k, pl.ds(g * LANES, LANES)] = (
                            hb[k, pl.ds(g * LANES, LANES)] * eeb)

                pltpu.async_copy(
                    hb, acc_sh.at[src_v.at[j]], ssems[q], add=True)

                @pl.when(j + 2 < NCH)
                def _():
                    issue_meta(j + 2, q)

    for b in range(NBUF):
        drain(ssems[b], b)

    plsc.subcore_barrier()

    pltpu.sync_copy(
        acc_sh.at[pl.ds(sid * rows_per_tile, rows_per_tile)],
        out_hbm.at[cid, pl.ds(sid * rows_per_tile, rows_per_tile)])


def _agg_kernel(h, src_c, dst_c, ee_c, EPT, NP, D, NCH):
    mesh = plsc.VectorSubcoreMesh(
        core_axis_name="c", subcore_axis_name="s",
        num_cores=NC, num_subcores=NS)
    kern = pl.kernel(
        functools.partial(_agg_body, EPT, NP, D, NCH),
        out_type=jax.ShapeDtypeStruct((NC, NP, D), jnp.float32),
        mesh=mesh,
        compiler_params=pltpu.CompilerParams(needs_layout_passes=False),
        scratch_types=(
            [
                pltpu.VMEM((NCH, CHUNK), jnp.int32),
                pltpu.VMEM((1, CHUNK), jnp.int32),
                pltpu.VMEM((1, CHUNK), jnp.int32),
                pltpu.VMEM((1, CHUNK), jnp.float32),
                pltpu.VMEM((1, CHUNK), jnp.float32),
            ]
            + [pltpu.VMEM((CHUNK, D), jnp.float32) for _ in range(NBUF)]
            + [pltpu.SemaphoreType.DMA for _ in range(4 * NBUF)]
            + [pltpu.VMEM_SHARED((NP, D), jnp.float32)]
        ),
    )
    return kern(h, src_c, dst_c, ee_c)


def _combine_body(p_ref, r_ref, o_ref):
    p = p_ref[...]
    r = r_ref[...]
    tot = p[0] + p[1]
    rr = r[0] + r[1]
    v = tot / rr
    o_ref[...] = jnp.where(v > 0.0, v, jnp.exp(jnp.minimum(v, 0.0)) - 1.0)


def _combine(parts, rs3, NP, D):
    BLK = 512
    return pl.pallas_call(
        _combine_body,
        grid=(NP // BLK,),
        in_specs=[
            pl.BlockSpec((NC, BLK, D), lambda i: (0, i, 0)),
            pl.BlockSpec((NC, BLK, 1), lambda i: (0, i, 0)),
        ],
        out_specs=pl.BlockSpec((BLK, D), lambda i: (i, 0)),
        out_shape=jax.ShapeDtypeStruct((NP, D), jnp.float32),
    )(parts, rs3)


def kernel(input, edge, W, a):
    N, DIN = input.shape
    D = W.shape[1]
    E = edge.shape[1]

    NP = ((N + 511) // 512) * 512
    NR = NP // 128
    EPT = ((E + NW * CHUNK - 1) // (NW * CHUNK)) * CHUNK
    NCH = EPT // CHUNK

    x_p = jnp.pad(input, ((0, NP - N), (0, 0)))
    a2 = jnp.stack([a[0, :D], a[0, D:]], axis=1)

    pad = NW * EPT - E
    src_f = jnp.pad(edge[0], (0, pad)).reshape(NW, EPT)
    dst_f = jnp.pad(edge[1], (0, pad)).reshape(NW, EPT)
    src_c = src_f.reshape(NW, NCH, CHUNK)
    dst_c = dst_f.reshape(NW, NCH, 1, CHUNK)

    h, st = _project(x_p, W, a2, NP, D)
    ee, rsp = _ee_kernel(st, src_f, dst_f, E, EPT, NP, NR)
    ee_c = ee.reshape(NW, NCH, 1, CHUNK)
    parts = _agg_kernel(h, src_c, dst_c, ee_c, EPT, NP, D, NCH)
    rs3 = rsp.reshape(NC, NP, 1)
    out = _combine(parts, rs3, NP, D)
    return out[:N]


# speedup vs baseline: 8.8167x; 1.0080x over previous
import functools

import jax
import jax.numpy as jnp
from jax import lax
from jax.experimental import pallas as pl
from jax.experimental.pallas import tpu as pltpu
from jax.experimental.pallas import tpu_sc as plsc

ALPHA = 0.2
LANES = 16
NC = 2
NS = 16
NW = NC * NS
CHUNK = 128


def _proj_body(x_ref, w_ref, a2_ref, h_ref, st_ref):
    h = jnp.dot(x_ref[...], w_ref[...], preferred_element_type=jnp.float32)
    h_ref[...] = h
    st_ref[...] = lax.dot_general(
        a2_ref[...], h, (((0,), (1,)), ((), ())),
        preferred_element_type=jnp.float32)


def _project(x_p, W, a2, NP, D):
    BLK = 512
    grid = NP // BLK
    return pl.pallas_call(
        _proj_body,
        grid=(grid,),
        in_specs=[
            pl.BlockSpec((BLK, x_p.shape[1]), lambda i: (i, 0)),
            pl.BlockSpec((x_p.shape[1], D), lambda i: (0, 0)),
            pl.BlockSpec((D, 2), lambda i: (0, 0)),
        ],
        out_specs=[
            pl.BlockSpec((BLK, D), lambda i: (i, 0)),
            pl.BlockSpec((2, BLK), lambda i: (0, i)),
        ],
        out_shape=[
            jax.ShapeDtypeStruct((NP, D), jnp.float32),
            jax.ShapeDtypeStruct((2, NP), jnp.float32),
        ],
    )(x_p, W, a2)


def _ee_kernel(st, src_w, dst_w, E, EPT, NP, NR):
    mesh = plsc.VectorSubcoreMesh(
        core_axis_name="c", subcore_axis_name="s",
        num_cores=NC, num_subcores=NS)
    kern = pl.kernel(
        functools.partial(_ee_body2, E, EPT, NP, NR),
        out_type=(
            jax.ShapeDtypeStruct((NW, EPT), jnp.float32),
            jax.ShapeDtypeStruct((NC, NR, 128), jnp.float32),
        ),
        mesh=mesh,
        compiler_params=pltpu.CompilerParams(needs_layout_passes=False),
        scratch_types=[
            pltpu.VMEM((NP,), jnp.float32),
            pltpu.VMEM((NP,), jnp.float32),
            pltpu.VMEM((EPT,), jnp.int32),
            pltpu.VMEM((EPT,), jnp.int32),
            pltpu.VMEM((EPT,), jnp.float32),
            pltpu.VMEM((NR, 128), jnp.float32),
            pltpu.VMEM((NR,), jnp.int32),
            pltpu.VMEM_SHARED((NR, 128), jnp.float32),
        ],
    )
    return kern(st, src_w, dst_w)


def _ee_body2(E, EPT, NP, NR,
              st_hbm, src_hbm, dst_hbm, ee_hbm, rs_hbm,
              s_v, t_v, src_v, dst_v, ee_v, rsum_v, iota_v, rs_sh):
    cid = lax.axis_index("c")
    sid = lax.axis_index("s")
    wid = sid * NC + cid

    zero = jnp.zeros((LANES,), jnp.float32)

    @pl.loop(0, NR)
    def _(r):
        for g in range(128 // LANES):
            rsum_v[r, pl.ds(g * LANES, LANES)] = zero

    @pl.loop(0, NR // LANES)
    def _(g):
        iota_v[pl.ds(g * LANES, LANES)] = (
            g * LANES + lax.iota(jnp.int32, LANES))

    @pl.when(sid == 0)
    def _():
        pltpu.sync_copy(rsum_v, rs_sh)

    pltpu.sync_copy(st_hbm.at[0], s_v)
    pltpu.sync_copy(st_hbm.at[1], t_v)
    pltpu.sync_copy(src_hbm.at[wid], src_v)
    pltpu.sync_copy(dst_hbm.at[wid], dst_v)

    plsc.subcore_barrier()

    base = wid * EPT

    @pl.loop(0, EPT // LANES)
    def _(i):
        off = i * LANES
        srcv = src_v[pl.ds(off, LANES)]
        dstv = dst_v[pl.ds(off, LANES)]
        sv = plsc.load_gather(s_v, [srcv])
        tv = plsc.load_gather(t_v, [dstv])
        logit = sv + tv
        lrelu = jnp.where(logit >= 0.0, logit, ALPHA * logit)
        ee = jnp.exp(-lrelu)
        pos = base + off + lax.iota(jnp.int32, LANES)
        ee = jnp.where(pos < E, ee, 0.0)
        ee_v[pl.ds(off, LANES)] = ee
        rhi = lax.shift_right_logical(srcv, 7)
        rlo = jnp.bitwise_and(srcv, 127)
        plsc.addupdate_scatter(rsum_v, [rhi, rlo], ee)

    pltpu.sync_copy(ee_v, ee_hbm.at[wid])

    pltpu.sync_copy(rsum_v, rs_sh.at[iota_v], add=True)

    plsc.subcore_barrier()

    @pl.when(sid == 0)
    def _():
        pltpu.sync_copy(rs_sh, rs_hbm.at[cid])


NBUF = 2


def _agg_body(EPT, NP, D, NCH,
              h_hbm, srci_hbm, dsti_hbm, ee_hbm, out_hbm,
              src_v, db0, db1, eb0, eb1,
              hb0, hb1,
              m0, m1, e0, e1, g0, g1, s0, s1,
              acc_sh):
    cid = lax.axis_index("c")
    sid = lax.axis_index("s")
    wid = sid * NC + cid
    dbs = [db0, db1]
    ebs = [eb0, eb1]
    hbs = [hb0, hb1]
    msems = [m0, m1]
    esems = [e0, e1]
    gsems = [g0, g1]
    ssems = [s0, s1]

    rows_per_tile = NP // NS

    zero = jnp.zeros((LANES,), jnp.float32)

    @pl.loop(0, CHUNK)
    def _(r):
        for g in range(D // LANES):
            hb0[r, pl.ds(g * LANES, LANES)] = zero

    @pl.loop(0, rows_per_tile // CHUNK)
    def _(b):
        pltpu.sync_copy(
            hb0, acc_sh.at[pl.ds(sid * rows_per_tile + b * CHUNK, CHUNK)])

    pltpu.sync_copy(srci_hbm.at[wid], src_v)

    plsc.subcore_barrier()

    def issue_meta(j, b):
        pltpu.async_copy(dsti_hbm.at[wid, j], dbs[b], msems[b])
        pltpu.async_copy(ee_hbm.at[wid, j], ebs[b], esems[b])

    def wait_meta(b):
        pltpu.make_async_copy(dsti_hbm.at[0, 0], dbs[b], msems[b]).wait()
        pltpu.make_async_copy(ee_hbm.at[0, 0], ebs[b], esems[b]).wait()

    def issue_gather(j, b):
        pltpu.async_copy(h_hbm.at[dbs[b].at[0]], hbs[b], gsems[b])

    def drain(sem, b):
        pltpu.make_async_copy(h_hbm.at[pl.ds(0, CHUNK)], hbs[b], sem).wait()

    issue_meta(0, 0)
    issue_meta(1, 1)
    wait_meta(0)
    issue_gather(0, 0)

    @pl.loop(0, NCH)
    def _(j):
        for q in range(NBUF):
            @pl.when(lax.rem(j, 2) == q)
            def _():
                nq = 1 - q
                hb = hbs[q]
                eb = ebs[q]

                @pl.when(j + 1 < NCH)
                def _():
                    @pl.when(j >= 1)
                    def _():
                        drain(ssems[nq], nq)
                    wait_meta(nq)
                    issue_gather(j + 1, nq)

                drain(gsems[q], q)

                z16 = jnp.zeros((LANES,), jnp.int32)

                @pl.loop(0, CHUNK)
                def _(k):
                    eeb = plsc.load_gather(eb, [z16, k + z16])
                    for g in range(D // LANES):
                        hb[k, pl.ds(g * LANES, LANES)] = (
                            hb[k, pl.ds(g * LANES, LANES)] * eeb)

                pltpu.async_copy(
                    hb, acc_sh.at[src_v.at[j]], ssems[q], add=True)

                @pl.when(j + 2 < NCH)
                def _():
                    issue_meta(j + 2, q)

    for b in range(NBUF):
        drain(ssems[b], b)

    plsc.subcore_barrier()

    pltpu.sync_copy(
        acc_sh.at[pl.ds(sid * rows_per_tile, rows_per_tile)],
        out_hbm.at[cid, pl.ds(sid * rows_per_tile, rows_per_tile)])


def _agg_kernel(h, src_c, dst_c, ee_c, EPT, NP, D, NCH):
    mesh = plsc.VectorSubcoreMesh(
        core_axis_name="c", subcore_axis_name="s",
        num_cores=NC, num_subcores=NS)
    kern = pl.kernel(
        functools.partial(_agg_body, EPT, NP, D, NCH),
        out_type=jax.ShapeDtypeStruct((NC, NP, D), jnp.float32),
        mesh=mesh,
        compiler_params=pltpu.CompilerParams(needs_layout_passes=False),
        scratch_types=(
            [
                pltpu.VMEM((NCH, CHUNK), jnp.int32),
                pltpu.VMEM((1, CHUNK), jnp.int32),
                pltpu.VMEM((1, CHUNK), jnp.int32),
                pltpu.VMEM((1, CHUNK), jnp.float32),
                pltpu.VMEM((1, CHUNK), jnp.float32),
            ]
            + [pltpu.VMEM((CHUNK, D), jnp.float32) for _ in range(NBUF)]
            + [pltpu.SemaphoreType.DMA for _ in range(4 * NBUF)]
            + [pltpu.VMEM_SHARED((NP, D), jnp.float32)]
        ),
    )
    return kern(h, src_c, dst_c, ee_c)


def _combine_body(p_ref, r_ref, o_ref):
    p = p_ref[...]
    r = r_ref[...]
    tot = p[0] + p[1]
    rr = r[0] + r[1]
    v = tot / rr
    o_ref[...] = jnp.where(v > 0.0, v, jnp.exp(jnp.minimum(v, 0.0)) - 1.0)


def _combine(parts, rs3, NP, D):
    BLK = 512
    return pl.pallas_call(
        _combine_body,
        grid=(NP // BLK,),
        in_specs=[
            pl.BlockSpec((NC, BLK, D), lambda i: (0, i, 0)),
            pl.BlockSpec((NC, BLK, 1), lambda i: (0, i, 0)),
        ],
        out_specs=pl.BlockSpec((BLK, D), lambda i: (i, 0)),
        out_shape=jax.ShapeDtypeStruct((NP, D), jnp.float32),
    )(parts, rs3)


def kernel(input, edge, W, a):
    N, DIN = input.shape
    D = W.shape[1]
    E = edge.shape[1]

    NP = ((N + 511) // 512) * 512
    NR = NP // 128
    EPT = ((E + NW * CHUNK - 1) // (NW * CHUNK)) * CHUNK
    NCH = EPT // CHUNK

    x_p = jnp.pad(input, ((0, NP - N), (0, 0)))
    a2 = jnp.stack([a[0, :D], a[0, D:]], axis=1)

    pad = NW * EPT - E
    pad_src = jnp.arange(pad, dtype=jnp.int32) % NP
    src_f = jnp.concatenate([edge[0], pad_src]).reshape(NW, EPT)
    dst_f = jnp.pad(edge[1], (0, pad)).reshape(NW, EPT)
    src_c = src_f.reshape(NW, NCH, CHUNK)
    dst_c = dst_f.reshape(NW, NCH, 1, CHUNK)

    h, st = _project(x_p, W, a2, NP, D)
    ee, rsp = _ee_kernel(st, src_f, dst_f, E, EPT, NP, NR)
    ee_c = ee.reshape(NW, NCH, 1, CHUNK)
    parts = _agg_kernel(h, src_c, dst_c, ee_c, EPT, NP, D, NCH)
    rs3 = rsp.reshape(NC, NP, 1)
    out = _combine(parts, rs3, NP, D)
    return out[:N]


# speedup vs baseline: 9.4615x; 1.0731x over previous
import functools

import jax
import jax.numpy as jnp
from jax import lax
from jax.experimental import pallas as pl
from jax.experimental.pallas import tpu as pltpu
from jax.experimental.pallas import tpu_sc as plsc

ALPHA = 0.2
LANES = 16
NC = 2
NS = 16
NW = NC * NS
CHUNK = 128


def _proj_body(x_ref, w_ref, a2_ref, h_ref, st_ref):
    h = jnp.dot(x_ref[...], w_ref[...], preferred_element_type=jnp.float32)
    h_ref[...] = h
    st_ref[...] = lax.dot_general(
        a2_ref[...], h, (((0,), (1,)), ((), ())),
        preferred_element_type=jnp.float32)


def _project(x_p, W, a2, NP, D):
    BLK = 512
    grid = NP // BLK
    return pl.pallas_call(
        _proj_body,
        grid=(grid,),
        in_specs=[
            pl.BlockSpec((BLK, x_p.shape[1]), lambda i: (i, 0)),
            pl.BlockSpec((x_p.shape[1], D), lambda i: (0, 0)),
            pl.BlockSpec((D, 2), lambda i: (0, 0)),
        ],
        out_specs=[
            pl.BlockSpec((BLK, D), lambda i: (i, 0)),
            pl.BlockSpec((2, BLK), lambda i: (0, i)),
        ],
        out_shape=[
            jax.ShapeDtypeStruct((NP, D), jnp.float32),
            jax.ShapeDtypeStruct((2, NP), jnp.float32),
        ],
    )(x_p, W, a2)


def _ee_kernel(st, src_w, dst_w, E, EPT, NP, NR):
    mesh = plsc.VectorSubcoreMesh(
        core_axis_name="c", subcore_axis_name="s",
        num_cores=NC, num_subcores=NS)
    kern = pl.kernel(
        functools.partial(_ee_body2, E, EPT, NP, NR),
        out_type=(
            jax.ShapeDtypeStruct((NW, EPT), jnp.float32),
            jax.ShapeDtypeStruct((NC, NR, 128), jnp.float32),
        ),
        mesh=mesh,
        compiler_params=pltpu.CompilerParams(needs_layout_passes=False),
        scratch_types=[
            pltpu.VMEM((NP,), jnp.float32),
            pltpu.VMEM((NP,), jnp.float32),
            pltpu.VMEM((EPT,), jnp.int32),
            pltpu.VMEM((EPT,), jnp.int32),
            pltpu.VMEM((EPT,), jnp.float32),
            pltpu.VMEM((NR, 128), jnp.float32),
            pltpu.VMEM((NR,), jnp.int32),
            pltpu.VMEM_SHARED((NR, 128), jnp.float32),
        ],
    )
    return kern(st, src_w, dst_w)


def _ee_body2(E, EPT, NP, NR,
              st_hbm, src_hbm, dst_hbm, ee_hbm, rs_hbm,
              s_v, t_v, src_v, dst_v, ee_v, rsum_v, iota_v, rs_sh):
    cid = lax.axis_index("c")
    sid = lax.axis_index("s")
    wid = sid * NC + cid

    zero = jnp.zeros((LANES,), jnp.float32)

    @pl.loop(0, NR)
    def _(r):
        for g in range(128 // LANES):
            rsum_v[r, pl.ds(g * LANES, LANES)] = zero

    @pl.loop(0, NR // LANES)
    def _(g):
        iota_v[pl.ds(g * LANES, LANES)] = (
            g * LANES + lax.iota(jnp.int32, LANES))

    @pl.when(sid == 0)
    def _():
        pltpu.sync_copy(rsum_v, rs_sh)

    pltpu.sync_copy(st_hbm.at[0], s_v)
    pltpu.sync_copy(st_hbm.at[1], t_v)
    pltpu.sync_copy(src_hbm.at[wid], src_v)
    pltpu.sync_copy(dst_hbm.at[wid], dst_v)

    plsc.subcore_barrier()

    base = wid * EPT

    @pl.loop(0, EPT // LANES)
    def _(i):
        off = i * LANES
        srcv = src_v[pl.ds(off, LANES)]
        dstv = dst_v[pl.ds(off, LANES)]
        sv = plsc.load_gather(s_v, [srcv])
        tv = plsc.load_gather(t_v, [dstv])
        logit = sv + tv
        lrelu = jnp.where(logit >= 0.0, logit, ALPHA * logit)
        ee = jnp.exp(-lrelu)
        pos = base + off + lax.iota(jnp.int32, LANES)
        ee = jnp.where(pos < E, ee, 0.0)
        ee_v[pl.ds(off, LANES)] = ee
        rhi = lax.shift_right_logical(srcv, 7)
        rlo = jnp.bitwise_and(srcv, 127)
        plsc.addupdate_scatter(rsum_v, [rhi, rlo], ee)

    pltpu.sync_copy(ee_v, ee_hbm.at[wid])

    pltpu.sync_copy(rsum_v, rs_sh.at[iota_v], add=True)

    plsc.subcore_barrier()

    @pl.when(sid == 0)
    def _():
        pltpu.sync_copy(rs_sh, rs_hbm.at[cid])


NBUF = 2


def _agg_body(EPT, NP, D, NCH,
              h_hbm, srci_hbm, dsti_hbm, ee_hbm, out_hbm,
              src_v, db0, db1, eb0, eb1,
              hb0, hb1,
              m0, m1, e0, e1, g0, g1, s0, s1,
              acc_sh):
    cid = lax.axis_index("c")
    sid = lax.axis_index("s")
    wid = sid * NC + cid
    dbs = [db0, db1]
    ebs = [eb0, eb1]
    hbs = [hb0, hb1]
    msems = [m0, m1]
    esems = [e0, e1]
    gsems = [g0, g1]
    ssems = [s0, s1]

    rows_per_tile = NP // NS

    zero = jnp.zeros((LANES,), jnp.float32)

    @pl.loop(0, CHUNK)
    def _(r):
        for g in range(D // LANES):
            hb0[r, pl.ds(g * LANES, LANES)] = zero

    @pl.loop(0, rows_per_tile // CHUNK)
    def _(b):
        pltpu.sync_copy(
            hb0, acc_sh.at[pl.ds(sid * rows_per_tile + b * CHUNK, CHUNK)])

    pltpu.sync_copy(srci_hbm.at[wid], src_v)

    plsc.subcore_barrier()

    def issue_meta(j, b):
        pltpu.async_copy(dsti_hbm.at[wid, j], dbs[b], msems[b])
        pltpu.async_copy(ee_hbm.at[wid, j], ebs[b], esems[b])

    def wait_meta(b):
        pltpu.make_async_copy(dsti_hbm.at[0, 0], dbs[b], msems[b]).wait()
        pltpu.make_async_copy(ee_hbm.at[0, 0], ebs[b], esems[b]).wait()

    def issue_gather(j, b):
        pltpu.async_copy(h_hbm.at[dbs[b].at[0]], hbs[b], gsems[b])

    def drain(sem, b):
        pltpu.make_async_copy(h_hbm.at[pl.ds(0, CHUNK)], hbs[b], sem).wait()

    issue_meta(0, 0)
    issue_meta(1, 1)
    wait_meta(0)
    issue_gather(0, 0)

    @pl.loop(0, NCH)
    def _(j):
        for q in range(NBUF):
            @pl.when(lax.rem(j, 2) == q)
            def _():
                nq = 1 - q
                hb = hbs[q]
                eb = ebs[q]

                @pl.when(j + 1 < NCH)
                def _():
                    @pl.when(j >= 1)
                    def _():
                        drain(ssems[nq], nq)
                    wait_meta(nq)
                    issue_gather(j + 1, nq)

                drain(gsems[q], q)

                @pl.loop(0, CHUNK, step=LANES)
                def _(k0):
                    eev = eb[0, pl.ds(k0, LANES)]
                    for u in range(LANES):
                        lane = jnp.full((LANES,), u, jnp.int32)
                        eeb = eev[lane]
                        r = k0 + u
                        for g in range(D // LANES):
                            hb[r, pl.ds(g * LANES, LANES)] = (
                                hb[r, pl.ds(g * LANES, LANES)] * eeb)

                pltpu.async_copy(
                    hb, acc_sh.at[src_v.at[j]], ssems[q], add=True)

                @pl.when(j + 2 < NCH)
                def _():
                    issue_meta(j + 2, q)

    for b in range(NBUF):
        drain(ssems[b], b)

    plsc.subcore_barrier()

    pltpu.sync_copy(
        acc_sh.at[pl.ds(sid * rows_per_tile, rows_per_tile)],
        out_hbm.at[cid, pl.ds(sid * rows_per_tile, rows_per_tile)])


def _agg_kernel(h, src_c, dst_c, ee_c, EPT, NP, D, NCH):
    mesh = plsc.VectorSubcoreMesh(
        core_axis_name="c", subcore_axis_name="s",
        num_cores=NC, num_subcores=NS)
    kern = pl.kernel(
        functools.partial(_agg_body, EPT, NP, D, NCH),
        out_type=jax.ShapeDtypeStruct((NC, NP, D), jnp.float32),
        mesh=mesh,
        compiler_params=pltpu.CompilerParams(needs_layout_passes=False),
        scratch_types=(
            [
                pltpu.VMEM((NCH, CHUNK), jnp.int32),
                pltpu.VMEM((1, CHUNK), jnp.int32),
                pltpu.VMEM((1, CHUNK), jnp.int32),
                pltpu.VMEM((1, CHUNK), jnp.float32),
                pltpu.VMEM((1, CHUNK), jnp.float32),
            ]
            + [pltpu.VMEM((CHUNK, D), jnp.float32) for _ in range(NBUF)]
            + [pltpu.SemaphoreType.DMA for _ in range(4 * NBUF)]
            + [pltpu.VMEM_SHARED((NP, D), jnp.float32)]
        ),
    )
    return kern(h, src_c, dst_c, ee_c)


def _combine_body(p_ref, r_ref, o_ref):
    p = p_ref[...]
    r = r_ref[...]
    tot = p[0] + p[1]
    rr = r[0] + r[1]
    v = tot / rr
    o_ref[...] = jnp.where(v > 0.0, v, jnp.exp(jnp.minimum(v, 0.0)) - 1.0)


def _combine(parts, rs3, NP, D):
    BLK = 512
    return pl.pallas_call(
        _combine_body,
        grid=(NP // BLK,),
        in_specs=[
            pl.BlockSpec((NC, BLK, D), lambda i: (0, i, 0)),
            pl.BlockSpec((NC, BLK, 1), lambda i: (0, i, 0)),
        ],
        out_specs=pl.BlockSpec((BLK, D), lambda i: (i, 0)),
        out_shape=jax.ShapeDtypeStruct((NP, D), jnp.float32),
    )(parts, rs3)


def kernel(input, edge, W, a):
    N, DIN = input.shape
    D = W.shape[1]
    E = edge.shape[1]

    NP = ((N + 511) // 512) * 512
    NR = NP // 128
    EPT = ((E + NW * CHUNK - 1) // (NW * CHUNK)) * CHUNK
    NCH = EPT // CHUNK

    x_p = jnp.pad(input, ((0, NP - N), (0, 0)))
    a2 = jnp.stack([a[0, :D], a[0, D:]], axis=1)

    pad = NW * EPT - E
    pad_src = jnp.arange(pad, dtype=jnp.int32) % NP
    src_f = jnp.concatenate([edge[0], pad_src]).reshape(NW, EPT)
    dst_f = jnp.pad(edge[1], (0, pad)).reshape(NW, EPT)
    src_c = src_f.reshape(NW, NCH, CHUNK)
    dst_c = dst_f.reshape(NW, NCH, 1, CHUNK)

    h, st = _project(x_p, W, a2, NP, D)
    ee, rsp = _ee_kernel(st, src_f, dst_f, E, EPT, NP, NR)
    ee_c = ee.reshape(NW, NCH, 1, CHUNK)
    parts = _agg_kernel(h, src_c, dst_c, ee_c, EPT, NP, D, NCH)
    rs3 = rsp.reshape(NC, NP, 1)
    out = _combine(parts, rs3, NP, D)
    return out[:N]


# speedup vs baseline: 14.9322x; 1.5782x over previous
import functools

import jax
import jax.numpy as jnp
from jax import lax
from jax.experimental import pallas as pl
from jax.experimental.pallas import tpu as pltpu
from jax.experimental.pallas import tpu_sc as plsc

ALPHA = 0.2
LANES = 16
NC = 2
NS = 16
NW = NC * NS
CHUNK = 128


def _proj_body(x_ref, w_ref, a2_ref, h_ref, st_ref):
    h = jnp.dot(x_ref[...], w_ref[...], preferred_element_type=jnp.float32)
    h_ref[...] = h
    st_ref[...] = lax.dot_general(
        a2_ref[...], h, (((0,), (1,)), ((), ())),
        preferred_element_type=jnp.float32)


def _project(x_p, W, a2, NP, D):
    BLK = 512
    grid = NP // BLK
    return pl.pallas_call(
        _proj_body,
        grid=(grid,),
        in_specs=[
            pl.BlockSpec((BLK, x_p.shape[1]), lambda i: (i, 0)),
            pl.BlockSpec((x_p.shape[1], D), lambda i: (0, 0)),
            pl.BlockSpec((D, 2), lambda i: (0, 0)),
        ],
        out_specs=[
            pl.BlockSpec((BLK, D), lambda i: (i, 0)),
            pl.BlockSpec((2, BLK), lambda i: (0, i)),
        ],
        out_shape=[
            jax.ShapeDtypeStruct((NP, D), jnp.float32),
            jax.ShapeDtypeStruct((2, NP), jnp.float32),
        ],
    )(x_p, W, a2)


def _ee_kernel(st, src_w, dst_w, E, EPT, NP, NR):
    mesh = plsc.VectorSubcoreMesh(
        core_axis_name="c", subcore_axis_name="s",
        num_cores=NC, num_subcores=NS)
    kern = pl.kernel(
        functools.partial(_ee_body2, E, EPT, NP, NR),
        out_type=(
            jax.ShapeDtypeStruct((NW, EPT), jnp.float32),
            jax.ShapeDtypeStruct((NC, NR, 128), jnp.float32),
        ),
        mesh=mesh,
        compiler_params=pltpu.CompilerParams(needs_layout_passes=False),
        scratch_types=[
            pltpu.VMEM((NP,), jnp.float32),
            pltpu.VMEM((NP,), jnp.float32),
            pltpu.VMEM((EPT,), jnp.int32),
            pltpu.VMEM((EPT,), jnp.int32),
            pltpu.VMEM((EPT,), jnp.float32),
            pltpu.VMEM((NR, 128), jnp.float32),
            pltpu.VMEM((NR,), jnp.int32),
            pltpu.VMEM_SHARED((NR, 128), jnp.float32),
        ],
    )
    return kern(st, src_w, dst_w)


def _ee_body2(E, EPT, NP, NR,
              st_hbm, src_hbm, dst_hbm, ee_hbm, rs_hbm,
              s_v, t_v, src_v, dst_v, ee_v, rsum_v, iota_v, rs_sh):
    cid = lax.axis_index("c")
    sid = lax.axis_index("s")
    wid = sid * NC + cid

    zero = jnp.zeros((LANES,), jnp.float32)

    @pl.loop(0, NR)
    def _(r):
        for g in range(128 // LANES):
            rsum_v[r, pl.ds(g * LANES, LANES)] = zero

    @pl.loop(0, NR // LANES)
    def _(g):
        iota_v[pl.ds(g * LANES, LANES)] = (
            g * LANES + lax.iota(jnp.int32, LANES))

    @pl.when(sid == 0)
    def _():
        pltpu.sync_copy(rsum_v, rs_sh)

    pltpu.sync_copy(st_hbm.at[0], s_v)
    pltpu.sync_copy(st_hbm.at[1], t_v)
    pltpu.sync_copy(src_hbm.at[wid], src_v)
    pltpu.sync_copy(dst_hbm.at[wid], dst_v)

    plsc.subcore_barrier()

    base = wid * EPT

    @pl.loop(0, EPT // LANES)
    def _(i):
        off = i * LANES
        srcv = src_v[pl.ds(off, LANES)]
        dstv = dst_v[pl.ds(off, LANES)]
        sv = plsc.load_gather(s_v, [srcv])
        tv = plsc.load_gather(t_v, [dstv])
        logit = sv + tv
        lrelu = jnp.where(logit >= 0.0, logit, ALPHA * logit)
        ee = jnp.exp(-lrelu)
        pos = base + off + lax.iota(jnp.int32, LANES)
        ee = jnp.where(pos < E, ee, 0.0)
        ee_v[pl.ds(off, LANES)] = ee
        rhi = lax.shift_right_logical(srcv, 7)
        rlo = jnp.bitwise_and(srcv, 127)
        plsc.addupdate_scatter(rsum_v, [rhi, rlo], ee)

    pltpu.sync_copy(ee_v, ee_hbm.at[wid])

    pltpu.sync_copy(rsum_v, rs_sh.at[iota_v], add=True)

    plsc.subcore_barrier()

    @pl.when(sid == 0)
    def _():
        pltpu.sync_copy(rs_sh, rs_hbm.at[cid])


NBUF = 2


def _agg_body(EPT, NP, D, NCH,
              h_hbm, srci_hbm, dsti_hbm, ee_hbm, out_hbm,
              src_v, db0, db1, eb0, eb1,
              hb0, hb1,
              m0, m1, e0, e1, g0, g1, s0, s1,
              acc_sh):
    cid = lax.axis_index("c")
    sid = lax.axis_index("s")
    wid = sid * NC + cid
    dbs = [db0, db1]
    ebs = [eb0, eb1]
    hbs = [hb0, hb1]
    msems = [m0, m1]
    esems = [e0, e1]
    gsems = [g0, g1]
    ssems = [s0, s1]

    rows_per_tile = NP // NS

    zero = jnp.zeros((LANES,), jnp.float32)

    @pl.loop(0, CHUNK)
    def _(r):
        for g in range(D // LANES):
            hb0[r, pl.ds(g * LANES, LANES)] = zero

    @pl.loop(0, rows_per_tile // CHUNK)
    def _(b):
        pltpu.sync_copy(
            hb0, acc_sh.at[pl.ds(sid * rows_per_tile + b * CHUNK, CHUNK)])

    pltpu.sync_copy(srci_hbm.at[wid], src_v)

    plsc.subcore_barrier()

    def issue_meta(j, b):
        pltpu.async_copy(dsti_hbm.at[wid, j], dbs[b], msems[b])
        pltpu.async_copy(ee_hbm.at[wid, j], ebs[b], esems[b])

    def wait_meta(b):
        pltpu.make_async_copy(dsti_hbm.at[0, 0], dbs[b], msems[b]).wait()
        pltpu.make_async_copy(ee_hbm.at[0, 0], ebs[b], esems[b]).wait()

    def issue_gather(j, b):
        pltpu.async_copy(h_hbm.at[dbs[b].at[0]], hbs[b], gsems[b])

    def drain(sem, b):
        pltpu.make_async_copy(h_hbm.at[pl.ds(0, CHUNK)], hbs[b], sem).wait()

    issue_meta(0, 0)
    issue_meta(1, 1)
    wait_meta(0)
    issue_gather(0, 0)

    @pl.loop(0, NCH)
    def _(j):
        for q in range(NBUF):
            @pl.when(lax.rem(j, 2) == q)
            def _():
                nq = 1 - q
                hb = hbs[q]
                eb = ebs[q]

                @pl.when(j + 1 < NCH)
                def _():
                    @pl.when(j >= 1)
                    def _():
                        drain(ssems[nq], nq)
                    wait_meta(nq)
                    issue_gather(j + 1, nq)

                drain(gsems[q], q)

                @pl.loop(0, CHUNK, step=LANES)
                def _(k0):
                    eev = eb[0, pl.ds(k0, LANES)]
                    for u in range(LANES):
                        lane = jnp.full((LANES,), u, jnp.int32)
                        eeb = eev[lane]
                        r = k0 + u
                        for g in range(D // LANES):
                            hb[r, pl.ds(g * LANES, LANES)] = (
                                hb[r, pl.ds(g * LANES, LANES)] * eeb)

                pltpu.async_copy(
                    hb, acc_sh.at[src_v.at[j]], ssems[q], add=True)

                @pl.when(j + 2 < NCH)
                def _():
                    issue_meta(j + 2, q)

    for b in range(NBUF):
        drain(ssems[b], b)

    plsc.subcore_barrier()

    pltpu.sync_copy(
        acc_sh.at[pl.ds(sid * rows_per_tile, rows_per_tile)],
        out_hbm.at[cid, pl.ds(sid * rows_per_tile, rows_per_tile)])


def _agg_kernel(h, src_c, dst_c, ee_c, EPT, NP, D, NCH):
    mesh = plsc.VectorSubcoreMesh(
        core_axis_name="c", subcore_axis_name="s",
        num_cores=NC, num_subcores=NS)
    kern = pl.kernel(
        functools.partial(_agg_body, EPT, NP, D, NCH),
        out_type=jax.ShapeDtypeStruct((NC, NP, D), jnp.float32),
        mesh=mesh,
        compiler_params=pltpu.CompilerParams(needs_layout_passes=False),
        scratch_types=(
            [
                pltpu.VMEM((NCH, CHUNK), jnp.int32),
                pltpu.VMEM((1, CHUNK), jnp.int32),
                pltpu.VMEM((1, CHUNK), jnp.int32),
                pltpu.VMEM((1, CHUNK), jnp.float32),
                pltpu.VMEM((1, CHUNK), jnp.float32),
            ]
            + [pltpu.VMEM((CHUNK, D), jnp.float32) for _ in range(NBUF)]
            + [pltpu.SemaphoreType.DMA for _ in range(4 * NBUF)]
            + [pltpu.VMEM_SHARED((NP, D), jnp.float32)]
        ),
    )
    return kern(h, src_c, dst_c, ee_c)


def _combine_body(p_ref, r_ref, o_ref):
    p = p_ref[...]
    r = r_ref[...]
    tot = p[0] + p[1]
    rr = r[0] + r[1]
    v = tot / rr
    o_ref[...] = jnp.where(v > 0.0, v, jnp.exp(jnp.minimum(v, 0.0)) - 1.0)


def _combine(parts, rs3, NP, D):
    BLK = 512
    return pl.pallas_call(
        _combine_body,
        grid=(NP // BLK,),
        in_specs=[
            pl.BlockSpec((NC, BLK, D), lambda i: (0, i, 0)),
            pl.BlockSpec((NC, BLK, 1), lambda i: (0, i, 0)),
        ],
        out_specs=pl.BlockSpec((BLK, D), lambda i: (i, 0)),
        out_shape=jax.ShapeDtypeStruct((NP, D), jnp.float32),
    )(parts, rs3)


def kernel(input, edge, W, a):
    N, DIN = input.shape
    D = W.shape[1]
    E = edge.shape[1]

    NP = ((N + 511) // 512) * 512
    NR = NP // 128
    EPT = ((E + NW * CHUNK - 1) // (NW * CHUNK)) * CHUNK
    NCH = EPT // CHUNK

    x_p = jnp.pad(input, ((0, NP - N), (0, 0)))
    a2 = jnp.stack([a[0, :D], a[0, D:]], axis=1)

    pad = NW * EPT - E
    pad_idx = jnp.arange(pad, dtype=jnp.int32) % NP
    src_f = jnp.concatenate([edge[0], pad_idx]).reshape(NW, EPT)
    dst_f = jnp.concatenate([edge[1], pad_idx]).reshape(NW, EPT)
    src_c = src_f.reshape(NW, NCH, CHUNK)
    dst_c = dst_f.reshape(NW, NCH, 1, CHUNK)

    h, st = _project(x_p, W, a2, NP, D)
    ee, rsp = _ee_kernel(st, src_f, dst_f, E, EPT, NP, NR)
    ee_c = ee.reshape(NW, NCH, 1, CHUNK)
    parts = _agg_kernel(h, src_c, dst_c, ee_c, EPT, NP, D, NCH)
    rs3 = rsp.reshape(NC, NP, 1)
    out = _combine(parts, rs3, NP, D)
    return out[:N]
